# baseline (speedup 1.0000x reference)
import functools

import jax
import jax.numpy as jnp
from jax import lax
from jax.experimental import pallas as pl
from jax.experimental.pallas import tpu as pltpu
from jax.experimental.pallas import tpu_sc as plsc

BUCKET_SIZE = 64
N_HASHES = 8
SEQLEN = 4096
NB = 64
LANES = 16
NVREG = SEQLEN // LANES


def _hash_body(w_ref, qkt_ref, b_ref):
    a = jnp.dot(w_ref[0], qkt_ref[0], preferred_element_type=jnp.float32)
    row = lax.broadcasted_iota(jnp.int32, a.shape, 0)
    mx = jnp.max(a, axis=0, keepdims=True)
    first_arg = jnp.min(jnp.where(a == mx, row, NB), axis=0, keepdims=True)
    b_ref[0, 0] = first_arg


def _hash_buckets(qkt, w):
    batch = qkt.shape[0]
    return pl.pallas_call(
        _hash_body,
        grid=(batch, N_HASHES),
        in_specs=[
            pl.BlockSpec((1, 64, 64), lambda b, h: (h, 0, 0)),
            pl.BlockSpec((1, 64, SEQLEN), lambda b, h: (b, 0, 0)),
        ],
        out_specs=pl.BlockSpec((1, 1, 1, SEQLEN), lambda b, h: (b, h, 0, 0)),
        out_shape=jax.ShapeDtypeStruct((batch, N_HASHES, 1, SEQLEN), jnp.int32),
    )(w, qkt).reshape(batch, N_HASHES, SEQLEN)


def _structure_body(bk_hbm, c0_hbm, c63_hbm, bk_v, c0_v, c63_v,
                    hist_v, off_v, offend_v):
    wid = lax.axis_index("s") * 2 + lax.axis_index("c")
    pltpu.sync_copy(bk_hbm.at[wid], bk_v)

    lane = lax.iota(jnp.int32, LANES)
    ones = jnp.ones((LANES,), jnp.int32)
    zeros = jnp.zeros((LANES,), jnp.int32)

    for j in range(LANES * NB // LANES):
        hist_v[pl.ds(j * LANES, LANES)] = zeros
    lane_base = lane * NB

    def hist_step(i, _):
        bkv = bk_v[pl.ds(i * LANES, LANES)]
        plsc.addupdate_scatter(hist_v, [lane_base + bkv], ones)
        return 0

    lax.fori_loop(0, NVREG, hist_step, 0)

    lo_bound = BUCKET_SIZE
    hi_bound = SEQLEN - BUCKET_SIZE
    run = jnp.int32(0)
    s0 = jnp.int32(0)
    lim0 = jnp.int32(0)
    sL = jnp.int32(0)
    limL = jnp.int32(0)
    for g in range(NB // LANES):
        cnt = zeros
        for l in range(LANES):
            cnt = cnt + hist_v[pl.ds(l * NB + g * LANES, LANES)]
        incl = plsc.cumsum(cnt)
        off = incl - cnt + run
        offend = off + cnt
        run = run + jnp.sum(cnt)
        off_v[pl.ds(g * LANES, LANES)] = off
        offend_v[pl.ds(g * LANES, LANES)] = offend
        binid = lane + g * LANES
        m0 = jnp.where((off < lo_bound) & (offend > lo_bound), 1, 0)
        s0 = s0 + jnp.sum(m0 * (binid + 1))
        lim0 = lim0 + jnp.sum(m0 * (lo_bound - off))
        mL = jnp.where((off < hi_bound) & (offend > hi_bound), 1, 0)
        sL = sL + jnp.sum(mL * (binid + 1))
        limL = limL + jnp.sum(mL * (hi_bound - off))
    s0v = jnp.full((LANES,), s0 - 1, jnp.int32)
    sLv = jnp.full((LANES,), sL - 1, jnp.int32)
    lim0v = jnp.full((LANES,), lim0, jnp.int32)
    limLv = jnp.full((LANES,), limL, jnp.int32)

    def sweep_step(i, carry):
        carry0, carryL = carry
        bkv = bk_v[pl.ds(i * LANES, LANES)]
        offv = plsc.load_gather(off_v, [bkv])
        offe = plsc.load_gather(offend_v, [bkv])
        is0 = bkv == s0v
        isL = bkv == sLv
        i0 = jnp.where(is0, 1, 0)
        iL = jnp.where(isL, 1, 0)
        r0 = plsc.cumsum(i0) - i0 + carry0
        rL = plsc.cumsum(iL) - iL + carryL
        c0 = (offe <= lo_bound) | (is0 & (r0 < lim0v))
        cL = (offv >= hi_bound) | (isL & (rL >= limLv))
        c0_v[pl.ds(i * LANES, LANES)] = jnp.where(c0, 1.0, 0.0)
        c63_v[pl.ds(i * LANES, LANES)] = jnp.where(cL, 1.0, 0.0)
        return (carry0 + jnp.sum(i0), carryL + jnp.sum(iL))

    lax.fori_loop(0, NVREG, sweep_step, (jnp.int32(0), jnp.int32(0)))

    pltpu.sync_copy(c0_v, c0_hbm.at[wid])
    pltpu.sync_copy(c63_v, c63_hbm.at[wid])


def _chunk_structure(buckets2):
    nrows = buckets2.shape[0]
    mesh = plsc.VectorSubcoreMesh(core_axis_name="c", subcore_axis_name="s")
    f = pl.kernel(
        _structure_body,
        out_type=[
            jax.ShapeDtypeStruct((nrows, SEQLEN), jnp.float32),
            jax.ShapeDtypeStruct((nrows, SEQLEN), jnp.float32),
        ],
        mesh=mesh,
        compiler_params=pltpu.CompilerParams(needs_layout_passes=False),
        scratch_types=[
            pltpu.VMEM((SEQLEN,), jnp.int32),
            pltpu.VMEM((SEQLEN,), jnp.float32),
            pltpu.VMEM((SEQLEN,), jnp.float32),
            pltpu.VMEM((LANES * NB,), jnp.int32),
            pltpu.VMEM((NB,), jnp.int32),
            pltpu.VMEM((NB,), jnp.int32),
        ],
    )
    return f(buckets2)


def _combine_body(v_ref, c0_ref, c63_ref, o_ref):
    m = 1.0 + c0_ref[...] * c63_ref[...]
    logits = jnp.log(m)
    mx = jnp.max(logits, axis=2, keepdims=True)
    lse = mx + jnp.log(jnp.sum(jnp.exp(logits - mx), axis=2, keepdims=True))
    w = jnp.sum(jnp.exp(logits - lse), axis=2, keepdims=True)
    o_ref[...] = v_ref[...] * w


def _combine(v, c0t, c63t):
    batch, seqlen, dim = v.shape
    tblk = 1024
    return pl.pallas_call(
        _combine_body,
        grid=(batch, seqlen // tblk),
        in_specs=[
            pl.BlockSpec((1, tblk, dim), lambda b, t: (b, t, 0)),
            pl.BlockSpec((1, tblk, N_HASHES), lambda b, t: (b, t, 0)),
            pl.BlockSpec((1, tblk, N_HASHES), lambda b, t: (b, t, 0)),
        ],
        out_specs=pl.BlockSpec((1, tblk, dim), lambda b, t: (b, t, 0)),
        out_shape=jax.ShapeDtypeStruct((batch, seqlen, dim), jnp.float32),
    )(v, c0t, c63t)


def kernel(qk, v):
    batch, seqlen, dim = qk.shape
    rot = jax.random.normal(jax.random.key(42),
                            (dim, N_HASHES, NB // 2), dtype=jnp.float32)
    rot2 = jnp.concatenate([rot, -rot], axis=2)
    w = jnp.transpose(rot2, (1, 2, 0))
    qkt = jnp.swapaxes(qk, 1, 2)
    buckets = _hash_buckets(qkt, w)
    c0, c63 = _chunk_structure(buckets.reshape(batch * N_HASHES, seqlen))
    c0 = c0.reshape(batch, N_HASHES, seqlen)
    c63 = c63.reshape(batch, N_HASHES, seqlen)
    c0t = jnp.swapaxes(c0, 1, 2)
    c63t = jnp.swapaxes(jnp.roll(c63, 1, axis=1), 1, 2)
    return _combine(v, c0t, c63t)


# speedup vs baseline: 1.4280x; 1.4280x over previous
import functools

import jax
import jax.numpy as jnp
from jax import lax
from jax.experimental import pallas as pl
from jax.experimental.pallas import tpu as pltpu
from jax.experimental.pallas import tpu_sc as plsc

BUCKET_SIZE = 64
N_HASHES = 8
SEQLEN = 4096
NB = 64
LANES = 16
NVREG = SEQLEN // LANES


def _hash_body(w_ref, qkt_ref, b_ref):
    a = jnp.dot(w_ref[...], qkt_ref[0], preferred_element_type=jnp.float32)
    a = a.reshape(N_HASHES, NB, -1)
    row = lax.broadcasted_iota(jnp.int32, a.shape, 1)
    mx = jnp.max(a, axis=1, keepdims=True)
    b_ref[...] = jnp.min(jnp.where(a == mx, row, NB), axis=1)


def _hash_buckets(qkt, w):
    batch = qkt.shape[0]
    nblk = 1024
    return pl.pallas_call(
        _hash_body,
        grid=(batch, SEQLEN // nblk),
        in_specs=[
            pl.BlockSpec((N_HASHES * NB, 64), lambda b, t: (0, 0)),
            pl.BlockSpec((1, 64, nblk), lambda b, t: (b, 0, t)),
        ],
        out_specs=pl.BlockSpec((N_HASHES, nblk), lambda b, t: (b, t)),
        out_shape=jax.ShapeDtypeStruct((batch * N_HASHES, SEQLEN), jnp.int32),
    )(w, qkt)


def _structure_body(bk_hbm, c0_hbm, c63_hbm, bk_v, c0_v, c63_v,
                    hist_v, off_v, offend_v):
    wid = lax.axis_index("s") * 2 + lax.axis_index("c")
    pltpu.sync_copy(bk_hbm.at[wid], bk_v)

    lane = lax.iota(jnp.int32, LANES)
    ones = jnp.ones((LANES,), jnp.int32)
    zeros = jnp.zeros((LANES,), jnp.int32)

    for j in range(LANES * NB // LANES):
        hist_v[pl.ds(j * LANES, LANES)] = zeros
    lane_base = lane * NB

    def hist_step(i, _):
        bkv = bk_v[pl.ds(i * LANES, LANES)]
        plsc.addupdate_scatter(hist_v, [lane_base + bkv], ones)
        return 0

    lax.fori_loop(0, NVREG, hist_step, 0)

    lo_bound = BUCKET_SIZE
    hi_bound = SEQLEN - BUCKET_SIZE
    run = jnp.int32(0)
    s0 = jnp.int32(0)
    lim0 = jnp.int32(0)
    sL = jnp.int32(0)
    limL = jnp.int32(0)
    for g in range(NB // LANES):
        cnt = zeros
        for l in range(LANES):
            cnt = cnt + hist_v[pl.ds(l * NB + g * LANES, LANES)]
        incl = plsc.cumsum(cnt)
        off = incl - cnt + run
        offend = off + cnt
        run = run + jnp.sum(cnt)
        off_v[pl.ds(g * LANES, LANES)] = off
        offend_v[pl.ds(g * LANES, LANES)] = offend
        binid = lane + g * LANES
        m0 = jnp.where((off < lo_bound) & (offend > lo_bound), 1, 0)
        s0 = s0 + jnp.sum(m0 * (binid + 1))
        lim0 = lim0 + jnp.sum(m0 * (lo_bound - off))
        mL = jnp.where((off < hi_bound) & (offend > hi_bound), 1, 0)
        sL = sL + jnp.sum(mL * (binid + 1))
        limL = limL + jnp.sum(mL * (hi_bound - off))
    s0v = jnp.full((LANES,), s0 - 1, jnp.int32)
    sLv = jnp.full((LANES,), sL - 1, jnp.int32)
    lim0v = jnp.full((LANES,), lim0, jnp.int32)
    limLv = jnp.full((LANES,), limL, jnp.int32)

    def sweep_step(i, carry):
        carry0, carryL = carry
        bkv = bk_v[pl.ds(i * LANES, LANES)]
        offv = plsc.load_gather(off_v, [bkv])
        offe = plsc.load_gather(offend_v, [bkv])
        is0 = bkv == s0v
        isL = bkv == sLv
        i0 = jnp.where(is0, 1, 0)
        iL = jnp.where(isL, 1, 0)
        r0 = plsc.cumsum(i0) - i0 + carry0
        rL = plsc.cumsum(iL) - iL + carryL
        c0 = (offe <= lo_bound) | (is0 & (r0 < lim0v))
        cL = (offv >= hi_bound) | (isL & (rL >= limLv))
        c0_v[pl.ds(i * LANES, LANES)] = jnp.where(c0, 1.0, 0.0)
        c63_v[pl.ds(i * LANES, LANES)] = jnp.where(cL, 1.0, 0.0)
        return (carry0 + jnp.sum(i0), carryL + jnp.sum(iL))

    lax.fori_loop(0, NVREG, sweep_step, (jnp.int32(0), jnp.int32(0)))

    pltpu.sync_copy(c0_v, c0_hbm.at[wid])
    pltpu.sync_copy(c63_v, c63_hbm.at[wid])


def _chunk_structure(buckets2):
    nrows = buckets2.shape[0]
    mesh = plsc.VectorSubcoreMesh(core_axis_name="c", subcore_axis_name="s")
    f = pl.kernel(
        _structure_body,
        out_type=[
            jax.ShapeDtypeStruct((nrows, SEQLEN), jnp.float32),
            jax.ShapeDtypeStruct((nrows, SEQLEN), jnp.float32),
        ],
        mesh=mesh,
        compiler_params=pltpu.CompilerParams(needs_layout_passes=False),
        scratch_types=[
            pltpu.VMEM((SEQLEN,), jnp.int32),
            pltpu.VMEM((SEQLEN,), jnp.float32),
            pltpu.VMEM((SEQLEN,), jnp.float32),
            pltpu.VMEM((LANES * NB,), jnp.int32),
            pltpu.VMEM((NB,), jnp.int32),
            pltpu.VMEM((NB,), jnp.int32),
        ],
    )
    return f(buckets2)


def _combine_body(v_ref, c0_ref, c63_ref, o_ref):
    c0 = c0_ref[...]
    c63 = c63_ref[...]
    c63r = jnp.concatenate([c63[N_HASHES - 1:], c63[:N_HASHES - 1]], axis=0)
    m = 1.0 + c0 * c63r
    logits = jnp.log(m)
    mx = jnp.max(logits, axis=0, keepdims=True)
    lse = mx + jnp.log(jnp.sum(jnp.exp(logits - mx), axis=0, keepdims=True))
    w = jnp.sum(jnp.exp(logits - lse), axis=0, keepdims=True)
    o_ref[0] = v_ref[0] * jnp.swapaxes(w, 0, 1)


def _combine(v, c0, c63):
    batch, seqlen, dim = v.shape
    tblk = 1024
    return pl.pallas_call(
        _combine_body,
        grid=(batch, seqlen // tblk),
        in_specs=[
            pl.BlockSpec((1, tblk, dim), lambda b, t: (b, t, 0)),
            pl.BlockSpec((N_HASHES, tblk), lambda b, t: (b, t)),
            pl.BlockSpec((N_HASHES, tblk), lambda b, t: (b, t)),
        ],
        out_specs=pl.BlockSpec((1, tblk, dim), lambda b, t: (b, t, 0)),
        out_shape=jax.ShapeDtypeStruct((batch, seqlen, dim), jnp.float32),
    )(v, c0, c63)


def kernel(qk, v):
    batch, seqlen, dim = qk.shape
    rot = jax.random.normal(jax.random.key(42),
                            (dim, N_HASHES, NB // 2), dtype=jnp.float32)
    rot2 = jnp.concatenate([rot, -rot], axis=2)
    w = jnp.transpose(rot2, (1, 2, 0)).reshape(N_HASHES * NB, dim)
    qkt = jnp.swapaxes(qk, 1, 2)
    buckets = _hash_buckets(qkt, w)
    c0, c63 = _chunk_structure(buckets)
    return _combine(v, c0, c63)


# speedup vs baseline: 1.4671x; 1.0274x over previous
import functools

import jax
import jax.numpy as jnp
from jax import lax
from jax.experimental import pallas as pl
from jax.experimental.pallas import tpu as pltpu
from jax.experimental.pallas import tpu_sc as plsc

BUCKET_SIZE = 64
N_HASHES = 8
SEQLEN = 4096
NB = 64
LANES = 16
NVREG = SEQLEN // LANES


def _hash_body(w_ref, qk_ref, b_ref):
    a = lax.dot_general(w_ref[...], qk_ref[0], (((1,), (1,)), ((), ())),
                        preferred_element_type=jnp.float32)
    a = a.reshape(N_HASHES, NB, -1)
    row = lax.broadcasted_iota(jnp.int32, a.shape, 1)
    mx = jnp.max(a, axis=1, keepdims=True)
    b_ref[...] = jnp.min(jnp.where(a == mx, row, NB), axis=1)


def _hash_buckets(qk, w):
    batch = qk.shape[0]
    nblk = 2048
    return pl.pallas_call(
        _hash_body,
        grid=(batch, SEQLEN // nblk),
        in_specs=[
            pl.BlockSpec((N_HASHES * NB, 64), lambda b, t: (0, 0)),
            pl.BlockSpec((1, nblk, 64), lambda b, t: (b, t, 0)),
        ],
        out_specs=pl.BlockSpec((N_HASHES, nblk), lambda b, t: (b, t)),
        out_shape=jax.ShapeDtypeStruct((batch * N_HASHES, SEQLEN), jnp.int32),
    )(w, qk)


def _structure_body(bk_hbm, c0_hbm, c63_hbm, bk_v, c0_v, c63_v,
                    hist_v, off_v, offend_v):
    wid = lax.axis_index("s") * 2 + lax.axis_index("c")
    pltpu.sync_copy(bk_hbm.at[wid], bk_v)

    lane = lax.iota(jnp.int32, LANES)
    ones = jnp.ones((LANES,), jnp.int32)
    zeros = jnp.zeros((LANES,), jnp.int32)

    for j in range(LANES * NB // LANES):
        hist_v[pl.ds(j * LANES, LANES)] = zeros
    lane_base = lane * NB

    def hist_step(i, _):
        for j in range(4):
            bkv = bk_v[pl.ds((i * 4 + j) * LANES, LANES)]
            plsc.addupdate_scatter(hist_v, [lane_base + bkv], ones)
        return 0

    lax.fori_loop(0, NVREG // 4, hist_step, 0)

    lo_bound = BUCKET_SIZE
    hi_bound = SEQLEN - BUCKET_SIZE
    run = jnp.int32(0)
    s0 = jnp.int32(0)
    lim0 = jnp.int32(0)
    sL = jnp.int32(0)
    limL = jnp.int32(0)
    for g in range(NB // LANES):
        cnt = zeros
        for l in range(LANES):
            cnt = cnt + hist_v[pl.ds(l * NB + g * LANES, LANES)]
        incl = plsc.cumsum(cnt)
        off = incl - cnt + run
        offend = off + cnt
        run = run + jnp.sum(cnt)
        off_v[pl.ds(g * LANES, LANES)] = off
        offend_v[pl.ds(g * LANES, LANES)] = offend
        binid = lane + g * LANES
        m0 = jnp.where((off < lo_bound) & (offend > lo_bound), 1, 0)
        s0 = s0 + jnp.sum(m0 * (binid + 1))
        lim0 = lim0 + jnp.sum(m0 * (lo_bound - off))
        mL = jnp.where((off < hi_bound) & (offend > hi_bound), 1, 0)
        sL = sL + jnp.sum(mL * (binid + 1))
        limL = limL + jnp.sum(mL * (hi_bound - off))
    s0v = jnp.full((LANES,), s0 - 1, jnp.int32)
    sLv = jnp.full((LANES,), sL - 1, jnp.int32)
    lim0v = jnp.full((LANES,), lim0, jnp.int32)
    limLv = jnp.full((LANES,), limL, jnp.int32)

    def sweep_step(i, carry):
        carry0, carryL = carry
        for j in range(4):
            bkv = bk_v[pl.ds((i * 4 + j) * LANES, LANES)]
            offv = plsc.load_gather(off_v, [bkv])
            offe = plsc.load_gather(offend_v, [bkv])
            is0 = bkv == s0v
            isL = bkv == sLv
            i0 = jnp.where(is0, 1, 0)
            iL = jnp.where(isL, 1, 0)
            r0 = plsc.cumsum(i0) - i0 + carry0
            rL = plsc.cumsum(iL) - iL + carryL
            c0 = (offe <= lo_bound) | (is0 & (r0 < lim0v))
            cL = (offv >= hi_bound) | (isL & (rL >= limLv))
            c0_v[pl.ds((i * 4 + j) * LANES, LANES)] = jnp.where(c0, 1.0, 0.0)
            c63_v[pl.ds((i * 4 + j) * LANES, LANES)] = jnp.where(cL, 1.0, 0.0)
            carry0 = carry0 + jnp.sum(i0)
            carryL = carryL + jnp.sum(iL)
        return (carry0, carryL)

    lax.fori_loop(0, NVREG // 4, sweep_step, (jnp.int32(0), jnp.int32(0)))

    pltpu.sync_copy(c0_v, c0_hbm.at[wid])
    pltpu.sync_copy(c63_v, c63_hbm.at[wid])


def _chunk_structure(buckets2):
    nrows = buckets2.shape[0]
    mesh = plsc.VectorSubcoreMesh(core_axis_name="c", subcore_axis_name="s")
    f = pl.kernel(
        _structure_body,
        out_type=[
            jax.ShapeDtypeStruct((nrows, SEQLEN), jnp.float32),
            jax.ShapeDtypeStruct((nrows, SEQLEN), jnp.float32),
        ],
        mesh=mesh,
        compiler_params=pltpu.CompilerParams(needs_layout_passes=False),
        scratch_types=[
            pltpu.VMEM((SEQLEN,), jnp.int32),
            pltpu.VMEM((SEQLEN,), jnp.float32),
            pltpu.VMEM((SEQLEN,), jnp.float32),
            pltpu.VMEM((LANES * NB,), jnp.int32),
            pltpu.VMEM((NB,), jnp.int32),
            pltpu.VMEM((NB,), jnp.int32),
        ],
    )
    return f(buckets2)


def _combine_body(v_ref, c0_ref, c63_ref, o_ref):
    c0 = c0_ref[...]
    c63 = c63_ref[...]
    c63r = jnp.concatenate([c63[N_HASHES - 1:], c63[:N_HASHES - 1]], axis=0)
    m = 1.0 + c0 * c63r
    logits = jnp.log(m)
    mx = jnp.max(logits, axis=0, keepdims=True)
    lse = mx + jnp.log(jnp.sum(jnp.exp(logits - mx), axis=0, keepdims=True))
    w = jnp.sum(jnp.exp(logits - lse), axis=0, keepdims=True)
    o_ref[0] = v_ref[0] * jnp.swapaxes(w, 0, 1)


def _combine(v, c0, c63):
    batch, seqlen, dim = v.shape
    tblk = 4096
    return pl.pallas_call(
        _combine_body,
        grid=(batch, seqlen // tblk),
        in_specs=[
            pl.BlockSpec((1, tblk, dim), lambda b, t: (b, t, 0)),
            pl.BlockSpec((N_HASHES, tblk), lambda b, t: (b, t)),
            pl.BlockSpec((N_HASHES, tblk), lambda b, t: (b, t)),
        ],
        out_specs=pl.BlockSpec((1, tblk, dim), lambda b, t: (b, t, 0)),
        out_shape=jax.ShapeDtypeStruct((batch, seqlen, dim), jnp.float32),
    )(v, c0, c63)


def kernel(qk, v):
    batch, seqlen, dim = qk.shape
    rot = jax.random.normal(jax.random.key(42),
                            (dim, N_HASHES, NB // 2), dtype=jnp.float32)
    rot2 = jnp.concatenate([rot, -rot], axis=2)
    w = jnp.transpose(rot2, (1, 2, 0)).reshape(N_HASHES * NB, dim)
    buckets = _hash_buckets(qk, w)
    c0, c63 = _chunk_structure(buckets)
    return _combine(v, c0, c63)


# speedup vs baseline: 3.6303x; 2.4745x over previous
import functools

import jax
import jax.numpy as jnp
from jax import lax
from jax.experimental import pallas as pl
from jax.experimental.pallas import tpu as pltpu
from jax.experimental.pallas import tpu_sc as plsc

BUCKET_SIZE = 64
N_HASHES = 8
SEQLEN = 4096
NB = 64
LANES = 16
NVREG = SEQLEN // LANES


def _hash_body(w_ref, qk_ref, b_ref):
    a = lax.dot_general(w_ref[...], qk_ref[0], (((1,), (1,)), ((), ())),
                        preferred_element_type=jnp.float32)
    a = a.reshape(N_HASHES, NB, -1)
    row = lax.broadcasted_iota(jnp.int32, a.shape, 1)
    mx = jnp.max(a, axis=1, keepdims=True)
    b_ref[...] = jnp.min(jnp.where(a == mx, row, NB), axis=1)


def _hash_buckets(qk, w):
    batch = qk.shape[0]
    nblk = 2048
    return pl.pallas_call(
        _hash_body,
        grid=(batch, SEQLEN // nblk),
        in_specs=[
            pl.BlockSpec((N_HASHES * NB, 64), lambda b, t: (0, 0)),
            pl.BlockSpec((1, nblk, 64), lambda b, t: (b, t, 0)),
        ],
        out_specs=pl.BlockSpec((N_HASHES, nblk), lambda b, t: (b, t)),
        out_shape=jax.ShapeDtypeStruct((batch * N_HASHES, SEQLEN), jnp.int32),
    )(w, qk)


def _structure_body(bk_hbm, c0_hbm, c63_hbm, bk_v, c0_v, c63_v,
                    hist_v, off_v, offend_v):
    wid = lax.axis_index("s") * 2 + lax.axis_index("c")
    pltpu.sync_copy(bk_hbm.at[wid], bk_v)

    lane = lax.iota(jnp.int32, LANES)
    ones = jnp.ones((LANES,), jnp.int32)
    zeros = jnp.zeros((LANES,), jnp.int32)

    for j in range(LANES * NB // LANES):
        hist_v[pl.ds(j * LANES, LANES)] = zeros
    lane_base = lane * NB

    def hist_step(i, _):
        for j in range(4):
            bkv = bk_v[pl.ds((i * 4 + j) * LANES, LANES)]
            plsc.addupdate_scatter(hist_v, [lane_base + bkv], ones)
        return 0

    lax.fori_loop(0, NVREG // 4, hist_step, 0)

    lo_bound = BUCKET_SIZE
    hi_bound = SEQLEN - BUCKET_SIZE
    run = jnp.int32(0)
    s0 = jnp.int32(0)
    lim0 = jnp.int32(0)
    sL = jnp.int32(0)
    limL = jnp.int32(0)
    for g in range(NB // LANES):
        cnt = zeros
        for l in range(LANES):
            cnt = cnt + hist_v[pl.ds(l * NB + g * LANES, LANES)]
        incl = plsc.cumsum(cnt)
        off = incl - cnt + run
        offend = off + cnt
        run = run + jnp.sum(cnt)
        off_v[pl.ds(g * LANES, LANES)] = off
        offend_v[pl.ds(g * LANES, LANES)] = offend
        binid = lane + g * LANES
        m0 = jnp.where((off < lo_bound) & (offend > lo_bound), 1, 0)
        s0 = s0 + jnp.sum(m0 * (binid + 1))
        lim0 = lim0 + jnp.sum(m0 * (lo_bound - off))
        mL = jnp.where((off < hi_bound) & (offend > hi_bound), 1, 0)
        sL = sL + jnp.sum(mL * (binid + 1))
        limL = limL + jnp.sum(mL * (hi_bound - off))
    s0v = jnp.full((LANES,), s0 - 1, jnp.int32)
    sLv = jnp.full((LANES,), sL - 1, jnp.int32)
    lim0v = jnp.full((LANES,), lim0, jnp.int32)
    limLv = jnp.full((LANES,), limL, jnp.int32)

    def sweep_step(i, carry):
        carry0, carryL = carry
        for j in range(4):
            bkv = bk_v[pl.ds((i * 4 + j) * LANES, LANES)]
            offv = plsc.load_gather(off_v, [bkv])
            offe = plsc.load_gather(offend_v, [bkv])
            is0 = bkv == s0v
            isL = bkv == sLv
            i0 = jnp.where(is0, 1, 0)
            iL = jnp.where(isL, 1, 0)
            r0 = plsc.cumsum(i0) - i0 + carry0
            rL = plsc.cumsum(iL) - iL + carryL
            c0 = (offe <= lo_bound) | (is0 & (r0 < lim0v))
            cL = (offv >= hi_bound) | (isL & (rL >= limLv))
            c0_v[pl.ds((i * 4 + j) * LANES, LANES)] = jnp.where(c0, 1.0, 0.0)
            c63_v[pl.ds((i * 4 + j) * LANES, LANES)] = jnp.where(cL, 1.0, 0.0)
            carry0 = carry0 + jnp.sum(i0)
            carryL = carryL + jnp.sum(iL)
        return (carry0, carryL)

    lax.fori_loop(0, NVREG // 4, sweep_step, (jnp.int32(0), jnp.int32(0)))

    pltpu.sync_copy(c0_v, c0_hbm.at[wid])
    pltpu.sync_copy(c63_v, c63_hbm.at[wid])


def _chunk_structure(buckets2):
    nrows = buckets2.shape[0]
    mesh = plsc.VectorSubcoreMesh(core_axis_name="c", subcore_axis_name="s")
    f = pl.kernel(
        _structure_body,
        out_type=[
            jax.ShapeDtypeStruct((nrows, SEQLEN), jnp.float32),
            jax.ShapeDtypeStruct((nrows, SEQLEN), jnp.float32),
        ],
        mesh=mesh,
        compiler_params=pltpu.CompilerParams(needs_layout_passes=False),
        scratch_types=[
            pltpu.VMEM((SEQLEN,), jnp.int32),
            pltpu.VMEM((SEQLEN,), jnp.float32),
            pltpu.VMEM((SEQLEN,), jnp.float32),
            pltpu.VMEM((LANES * NB,), jnp.int32),
            pltpu.VMEM((NB,), jnp.int32),
            pltpu.VMEM((NB,), jnp.int32),
        ],
    )
    return f(buckets2)


def _combine_body(v_ref, c0_ref, c63_ref, o_ref):
    c0 = c0_ref[...]
    c63 = c63_ref[...]
    c63r = jnp.concatenate([c63[N_HASHES - 1:], c63[:N_HASHES - 1]], axis=0)
    m = 1.0 + c0 * c63r
    logits = jnp.log(m)
    mx = jnp.max(logits, axis=0, keepdims=True)
    lse = mx + jnp.log(jnp.sum(jnp.exp(logits - mx), axis=0, keepdims=True))
    w = jnp.sum(jnp.exp(logits - lse), axis=0, keepdims=True)
    o_ref[0] = v_ref[0] * jnp.swapaxes(w, 0, 1)


def _combine(v, c0, c63):
    batch, seqlen, dim = v.shape
    tblk = 4096
    return pl.pallas_call(
        _combine_body,
        grid=(batch, seqlen // tblk),
        in_specs=[
            pl.BlockSpec((1, tblk, dim), lambda b, t: (b, t, 0)),
            pl.BlockSpec((N_HASHES, tblk), lambda b, t: (b, t)),
            pl.BlockSpec((N_HASHES, tblk), lambda b, t: (b, t)),
        ],
        out_specs=pl.BlockSpec((1, tblk, dim), lambda b, t: (b, t, 0)),
        out_shape=jax.ShapeDtypeStruct((batch, seqlen, dim), jnp.float32),
    )(v, c0, c63)


def kernel(qk, v):
    batch, seqlen, dim = qk.shape
    rot = jax.random.normal(jax.random.key(42),
                            (dim, N_HASHES, NB // 2), dtype=jnp.float32)
    rot2 = jnp.concatenate([rot, -rot], axis=2)
    w = jnp.transpose(rot2, (1, 2, 0)).reshape(N_HASHES * NB, dim)
    buckets = _hash_buckets(qk, w)
    c0, c63 = _chunk_structure(buckets)
    c0 = jnp.zeros_like(c0); c63 = jnp.zeros_like(c63)
    return _combine(v, c0, c63)
